# Initial kernel scaffold; baseline (speedup 1.0000x reference)
import jax
import jax.numpy as jnp
from jax.experimental import pallas as pl


def kernel(x, Wr, br, W1, b1, W2, b2):
    raise NotImplementedError("write your pallas kernel here")


# speedup vs baseline: 1.5192x; 1.5192x over previous
import functools

import jax
import jax.numpy as jnp
from jax.experimental import pallas as pl
from jax.experimental.pallas import tpu as pltpu

_E = 16
_TOPK = 2
_BM = 256


def _ffn_body(nact_ref, xidx_ref, bmap_ref, x_ref, w1_ref, b1_ref, w2_ref,
              b2_ref, wcol_ref, o_ref):
    g = pl.program_id(0)

    @pl.when(g < nact_ref[0])
    def _():
        hmid = jnp.dot(x_ref[...], w1_ref[0],
                       preferred_element_type=jnp.float32)
        hmid = jnp.maximum(hmid + b1_ref[0], 0.0)
        y = jnp.dot(hmid, w2_ref[0], preferred_element_type=jnp.float32)
        y = y + b2_ref[0]
        o_ref[...] = y * wcol_ref[...]


def _grouped_ffn(nact, xidx, bmap, xs, W1, b1, W2, b2, w_col, NB, P, H, F):
    grid_spec = pltpu.PrefetchScalarGridSpec(
        num_scalar_prefetch=3,
        grid=(NB,),
        in_specs=[
            pl.BlockSpec((_BM, H), lambda g, n, xi, bm: (xi[g], 0)),
            pl.BlockSpec((1, H, F), lambda g, n, xi, bm: (bm[g], 0, 0)),
            pl.BlockSpec((1, 1, F), lambda g, n, xi, bm: (bm[g], 0, 0)),
            pl.BlockSpec((1, F, H), lambda g, n, xi, bm: (bm[g], 0, 0)),
            pl.BlockSpec((1, 1, H), lambda g, n, xi, bm: (bm[g], 0, 0)),
            pl.BlockSpec((_BM, 1), lambda g, n, xi, bm: (xi[g], 0)),
        ],
        out_specs=pl.BlockSpec((_BM, H), lambda g, n, xi, bm: (xi[g], 0)),
    )
    return pl.pallas_call(
        _ffn_body,
        grid_spec=grid_spec,
        out_shape=jax.ShapeDtypeStruct((P, H), jnp.float32),
    )(nact, xidx, bmap, xs, W1, b1, W2, b2, w_col)


def kernel(x, Wr, br, W1, b1, W2, b2):
    b, s, h = x.shape
    T = b * s
    F = W1.shape[-1]
    E = Wr.shape[-1]
    xf = x.reshape(T, h)

    logits = xf @ Wr + br
    probs = jax.nn.softmax(logits, axis=-1)
    topw, topi = jax.lax.top_k(probs, _TOPK)

    e_flat = topi.T.reshape(-1).astype(jnp.int32)
    w_flat = topw.T.reshape(-1)
    tok = jnp.tile(jnp.arange(T, dtype=jnp.int32), _TOPK)

    onehot = (e_flat[:, None] == jnp.arange(E, dtype=jnp.int32)[None, :]
              ).astype(jnp.int32)
    ranks_all = jnp.cumsum(onehot, axis=0) - onehot
    rank = jnp.sum(ranks_all * onehot, axis=1)
    counts = jnp.sum(onehot, axis=0)
    blocks = (counts + _BM - 1) // _BM
    bstart = jnp.cumsum(blocks) - blocks
    seg_start = _BM * bstart
    pos = seg_start[e_flat] + rank

    NB = (_TOPK * T) // _BM + E
    P = NB * _BM
    tok_sorted = jnp.zeros((P,), jnp.int32).at[pos].set(tok)
    w_sorted = jnp.zeros((P,), x.dtype).at[pos].set(w_flat)
    nact = jnp.sum(blocks).astype(jnp.int32)

    gidx = jnp.arange(NB, dtype=jnp.int32)
    bmap = jnp.sum(gidx[:, None] >= bstart[None, :], axis=1).astype(
        jnp.int32) - 1
    last = bmap[nact - 1]
    bmap = jnp.where(gidx < nact, bmap, last)
    xidx = jnp.where(gidx < nact, gidx, nact - 1).astype(jnp.int32)

    xs = xf[tok_sorted]

    ys = _grouped_ffn(nact[None], xidx, bmap, xs, W1, b1[:, None, :], W2,
                      b2[:, None, :], w_sorted[:, None], NB, P, h, F)

    out = ys[pos[:T]] + ys[pos[T:]]
    return out.reshape(b, s, h)
